# baseline (speedup 1.0000x reference)
import functools

import jax
import jax.numpy as jnp
from jax import lax
from jax.experimental import pallas as pl
from jax.experimental.pallas import tpu as pltpu
from jax.experimental.pallas import tpu_sc as plsc

_B, _S, _WL = 32, 512, 16
_WDIM, _CDIM, _F, _W = 128, 64, 64, 5
_HID = _WDIM + _F
_NWORDS = _B * _S
_NPOS = _WL - _W + 1

_NC, _NS = 2, 16
_NWK = _NC * _NS
_CHUNK = 128


@functools.cache
def _get_sc_gather(nwords):
    nchars = nwords * _WL
    wch = nwords // (_NWK * _CHUNK)
    cch = nchars // (_NWK * _CHUNK)
    ngrp = cch // 4

    def body(wtbl, widx, ctbl, cidx, wout, cout,
             widx_v, cidx_v, wbuf, cbufs, wsem, cgs, css):
        wid = lax.axis_index("s") * _NC + lax.axis_index("c")
        wbase = wid * (wch * _CHUNK)
        cbase = wid * (cch * _CHUNK)
        pltpu.sync_copy(widx.at[pl.ds(wbase, wch * _CHUNK)], widx_v)
        pltpu.sync_copy(cidx.at[pl.ds(cbase, cch * _CHUNK)], cidx_v)

        def cgather(j, b):
            return pltpu.make_async_copy(
                ctbl.at[cidx_v.at[pl.ds(j * _CHUNK, _CHUNK)]], cbufs.at[b],
                cgs.at[b])

        def cstore(j, b):
            return pltpu.make_async_copy(
                cbufs.at[b], cout.at[pl.ds(cbase + j * _CHUNK, _CHUNK)],
                css.at[b])

        for b in range(4):
            cgather(b, b).start()
        for b in range(4):
            cgather(b, b).wait()
            cstore(b, b).start()
        for b in range(4):
            cgather(4 + b, 4 + b).start()

        def grp(g, carry):
            cur = 4 * (g % 2)
            oth = 4 * ((g + 1) % 2)
            for b in range(4):
                cstore(4 * (g - 1) + b, oth + b).wait()
                cgather(4 * (g + 1) + b, oth + b).start()
            for b in range(4):
                cgather(4 * g + b, cur + b).wait()
                cstore(4 * g + b, cur + b).start()
            return carry

        lax.fori_loop(1, ngrp - 1, grp, 0)

        glast = ngrp - 1
        gl = 4 * (glast % 2)
        for b in range(4):
            cgather(4 * glast + b, gl + b).wait()
            cstore(4 * glast + b, gl + b).start()
        for b in range(4):
            cstore(4 * (glast - 1) + b, (4 - gl) + b).wait()
            cstore(4 * glast + b, gl + b).wait()

        def wbody(j, carry):
            pltpu.async_copy(wtbl.at[widx_v.at[pl.ds(j * _CHUNK, _CHUNK)]],
                             wbuf, wsem).wait()
            pltpu.sync_copy(wbuf, wout.at[pl.ds(wbase + j * _CHUNK, _CHUNK)])
            return carry

        lax.fori_loop(0, wch, wbody, 0)

    return pl.kernel(
        body,
        out_type=[
            jax.ShapeDtypeStruct((nwords, _WDIM), jnp.float32),
            jax.ShapeDtypeStruct((nchars, _CDIM), jnp.float32),
        ],
        mesh=plsc.VectorSubcoreMesh(core_axis_name="c", subcore_axis_name="s",
                                    num_cores=_NC, num_subcores=_NS),
        scratch_types=[
            pltpu.VMEM((wch * _CHUNK,), jnp.int32),
            pltpu.VMEM((cch * _CHUNK,), jnp.int32),
            pltpu.VMEM((_CHUNK, _WDIM), jnp.float32),
            pltpu.VMEM((8, _CHUNK, _CDIM), jnp.float32),
            pltpu.SemaphoreType.DMA,
            pltpu.SemaphoreType.DMA((8,)),
            pltpu.SemaphoreType.DMA((8,)),
        ],
        compiler_params=pltpu.CompilerParams(use_tc_tiling_on_sc=False),
    )


_M = 512


def _tc_dense_body(ce_ref, x_ref, wr_ref, unk_ref, pwt_ref, wc_ref, cb_ref,
                   gwt0_ref, gb0_ref, twt0_ref, tb0_ref,
                   gwt1_ref, gb1_ref, twt1_ref, tb1_ref, o_ref):
    f32 = jnp.float32
    bf16 = jnp.bfloat16
    z = jnp.dot(ce_ref[...].astype(bf16), wc_ref[...],
                preferred_element_type=f32)
    cb = cb_ref[...]
    cm = jnp.maximum(z[:, 0:_F] + cb, 0.0)
    for t in range(1, _NPOS):
        cm = jnp.maximum(cm, jnp.maximum(z[:, t * _F:(t + 1) * _F] + cb, 0.0))
    mask = x_ref[...] == 1
    emb = jnp.where(mask, unk_ref[...], wr_ref[...])
    p = jnp.dot(emb.astype(bf16), pwt_ref[...], preferred_element_type=f32)
    h = jnp.concatenate([p, cm], axis=1)
    for gwt, gb, twt, tb in ((gwt0_ref, gb0_ref, twt0_ref, tb0_ref),
                             (gwt1_ref, gb1_ref, twt1_ref, tb1_ref)):
        hb = h.astype(bf16)
        g = jax.nn.sigmoid(jnp.dot(hb, gwt[...], preferred_element_type=f32)
                           + gb[...])
        t = jnp.maximum(jnp.dot(hb, twt[...], preferred_element_type=f32)
                        + tb[...], 0.0)
        h = g * t + (1.0 - g) * h
    o_ref[...] = h


def _full(shape):
    return pl.BlockSpec(shape, lambda i: (0, 0))


@functools.cache
def _get_tc_dense(nwords):
    return pl.pallas_call(
        _tc_dense_body,
        grid=(nwords // _M,),
        in_specs=[
            pl.BlockSpec((_M, _WL * _CDIM), lambda i: (i, 0)),
            pl.BlockSpec((_M, 1), lambda i: (i, 0)),
            pl.BlockSpec((_M, _WDIM), lambda i: (i, 0)),
            _full((1, _WDIM)),
            _full((_WDIM, _WDIM)),
            _full((_WL * _CDIM, _NPOS * _F)),
            _full((1, _F)),
            _full((_HID, _HID)), _full((1, _HID)),
            _full((_HID, _HID)), _full((1, _HID)),
            _full((_HID, _HID)), _full((1, _HID)),
            _full((_HID, _HID)), _full((1, _HID)),
        ],
        out_specs=pl.BlockSpec((_M, _HID), lambda i: (i, 0)),
        out_shape=jax.ShapeDtypeStruct((nwords, _HID), jnp.float32),
    )


def _conv_toeplitz(conv_w):
    kflat = jnp.transpose(conv_w, (2, 1, 0)).reshape(_W * _CDIM, _F)
    cols = [jnp.pad(kflat, ((_CDIM * t, _CDIM * (_NPOS - 1 - t)), (0, 0)))
            for t in range(_NPOS)]
    return jnp.concatenate(cols, axis=1)


_HALVES = 2


def kernel(x, c, word_table, unk_emb, proj_w, char_table, conv_w, conv_b,
           tw0, tb0, tw1, tb1, gw0, gb0, gw1, gb1):
    bf16 = jnp.bfloat16
    xf = x.astype(jnp.int32).reshape(-1)
    cf = c.astype(jnp.int32).reshape(-1)
    h = _NWORDS // _HALVES
    sc = _get_sc_gather(h)
    tc = _get_tc_dense(h)
    weights = (unk_emb, proj_w.T.astype(bf16),
               _conv_toeplitz(conv_w).astype(bf16), conv_b.reshape(1, _F),
               gw0.T.astype(bf16), gb0.reshape(1, _HID),
               tw0.T.astype(bf16), tb0.reshape(1, _HID),
               gw1.T.astype(bf16), gb1.reshape(1, _HID),
               tw1.T.astype(bf16), tb1.reshape(1, _HID))
    outs = []
    for i in range(_HALVES):
        xi = lax.dynamic_slice_in_dim(xf, i * h, h)
        ci = lax.dynamic_slice_in_dim(cf, i * h * _WL, h * _WL)
        wrows, crows = sc(word_table, xi, char_table, ci)
        outs.append(tc(crows.reshape(h, _WL * _CDIM), xi.reshape(-1, 1),
                       wrows, *weights))
    return jnp.concatenate(outs, axis=0).reshape(_B, _S, _HID)


# speedup vs baseline: 1.0718x; 1.0718x over previous
import functools

import jax
import jax.numpy as jnp
from jax import lax
from jax.experimental import pallas as pl
from jax.experimental.pallas import tpu as pltpu
from jax.experimental.pallas import tpu_sc as plsc

_B, _S, _WL = 32, 512, 16
_WDIM, _CDIM, _F, _W = 128, 64, 64, 5
_HID = _WDIM + _F
_NWORDS = _B * _S
_NPOS = _WL - _W + 1

_NC, _NS = 2, 16
_NWK = _NC * _NS
_CHUNK = 128


@functools.cache
def _get_sc_gather(nwords):
    nchars = nwords * _WL
    wch = nwords // (_NWK * _CHUNK)
    cch = nchars // (_NWK * _CHUNK)
    ngrp = cch // 4

    def body(wtbl, widx, ctbl, cidx, wout, cout,
             widx_v, cidx_v, wbufs, cbufs, wgs, wss, cgs, css):
        wid = lax.axis_index("s") * _NC + lax.axis_index("c")
        wbase = wid * (wch * _CHUNK)
        cbase = wid * (cch * _CHUNK)
        pltpu.sync_copy(widx.at[pl.ds(wbase, wch * _CHUNK)], widx_v)
        pltpu.sync_copy(cidx.at[pl.ds(cbase, cch * _CHUNK)], cidx_v)

        def wgather(j, b):
            return pltpu.make_async_copy(
                wtbl.at[widx_v.at[pl.ds(j * _CHUNK, _CHUNK)]], wbufs.at[b],
                wgs.at[b])

        def wstore(j, b):
            return pltpu.make_async_copy(
                wbufs.at[b], wout.at[pl.ds(wbase + j * _CHUNK, _CHUNK)],
                wss.at[b])

        wgather(0, 0).start()
        wgather(1, 1).start()

        def cgather(j, b):
            return pltpu.make_async_copy(
                ctbl.at[cidx_v.at[pl.ds(j * _CHUNK, _CHUNK)]], cbufs.at[b],
                cgs.at[b])

        def cstore(j, b):
            return pltpu.make_async_copy(
                cbufs.at[b], cout.at[pl.ds(cbase + j * _CHUNK, _CHUNK)],
                css.at[b])

        for b in range(4):
            cgather(b, b).start()
        for b in range(4):
            cgather(b, b).wait()
            cstore(b, b).start()
        for b in range(4):
            cgather(4 + b, 4 + b).start()

        def grp(g, carry):
            cur = 4 * (g % 2)
            oth = 4 * ((g + 1) % 2)
            for b in range(4):
                cstore(4 * (g - 1) + b, oth + b).wait()
                cgather(4 * (g + 1) + b, oth + b).start()
            for b in range(4):
                cgather(4 * g + b, cur + b).wait()
                cstore(4 * g + b, cur + b).start()
            return carry

        lax.fori_loop(1, ngrp - 1, grp, 0)

        glast = ngrp - 1
        gl = 4 * (glast % 2)
        for b in range(4):
            cgather(4 * glast + b, gl + b).wait()
            cstore(4 * glast + b, gl + b).start()
        for b in range(4):
            cstore(4 * (glast - 1) + b, (4 - gl) + b).wait()
            cstore(4 * glast + b, gl + b).wait()

        for j in range(wch):
            b = j % 2
            wgather(j, b).wait()
            wstore(j, b).start()
            if j + 2 < wch:
                wstore(j, b).wait()
                wgather(j + 2, b).start()
        for j in range(max(wch - 2, 0), wch):
            wstore(j, j % 2).wait()

    return pl.kernel(
        body,
        out_type=[
            jax.ShapeDtypeStruct((nwords, _WDIM), jnp.float32),
            jax.ShapeDtypeStruct((nchars, _CDIM), jnp.float32),
        ],
        mesh=plsc.VectorSubcoreMesh(core_axis_name="c", subcore_axis_name="s",
                                    num_cores=_NC, num_subcores=_NS),
        scratch_types=[
            pltpu.VMEM((wch * _CHUNK,), jnp.int32),
            pltpu.VMEM((cch * _CHUNK,), jnp.int32),
            pltpu.VMEM((2, _CHUNK, _WDIM), jnp.float32),
            pltpu.VMEM((8, _CHUNK, _CDIM), jnp.float32),
            pltpu.SemaphoreType.DMA((2,)),
            pltpu.SemaphoreType.DMA((2,)),
            pltpu.SemaphoreType.DMA((8,)),
            pltpu.SemaphoreType.DMA((8,)),
        ],
        compiler_params=pltpu.CompilerParams(use_tc_tiling_on_sc=False),
    )


_M = 512


def _tc_dense_body(ce_ref, x_ref, wr_ref, unk_ref, pwt_ref, wc_ref, cb_ref,
                   gwt0_ref, gb0_ref, twt0_ref, tb0_ref,
                   gwt1_ref, gb1_ref, twt1_ref, tb1_ref, o_ref):
    f32 = jnp.float32
    bf16 = jnp.bfloat16
    z = jnp.dot(ce_ref[...].astype(bf16), wc_ref[...],
                preferred_element_type=f32)
    cb = cb_ref[...]
    cm = jnp.maximum(z[:, 0:_F] + cb, 0.0)
    for t in range(1, _NPOS):
        cm = jnp.maximum(cm, jnp.maximum(z[:, t * _F:(t + 1) * _F] + cb, 0.0))
    mask = x_ref[...] == 1
    emb = jnp.where(mask, unk_ref[...], wr_ref[...])
    p = jnp.dot(emb.astype(bf16), pwt_ref[...], preferred_element_type=f32)
    h = jnp.concatenate([p, cm], axis=1)
    for gwt, gb, twt, tb in ((gwt0_ref, gb0_ref, twt0_ref, tb0_ref),
                             (gwt1_ref, gb1_ref, twt1_ref, tb1_ref)):
        hb = h.astype(bf16)
        g = jax.nn.sigmoid(jnp.dot(hb, gwt[...], preferred_element_type=f32)
                           + gb[...])
        t = jnp.maximum(jnp.dot(hb, twt[...], preferred_element_type=f32)
                        + tb[...], 0.0)
        h = g * t + (1.0 - g) * h
    o_ref[...] = h


def _full(shape):
    return pl.BlockSpec(shape, lambda i: (0, 0))


@functools.cache
def _get_tc_dense(nwords):
    return pl.pallas_call(
        _tc_dense_body,
        grid=(nwords // _M,),
        in_specs=[
            pl.BlockSpec((_M, _WL * _CDIM), lambda i: (i, 0)),
            pl.BlockSpec((_M, 1), lambda i: (i, 0)),
            pl.BlockSpec((_M, _WDIM), lambda i: (i, 0)),
            _full((1, _WDIM)),
            _full((_WDIM, _WDIM)),
            _full((_WL * _CDIM, _NPOS * _F)),
            _full((1, _F)),
            _full((_HID, _HID)), _full((1, _HID)),
            _full((_HID, _HID)), _full((1, _HID)),
            _full((_HID, _HID)), _full((1, _HID)),
            _full((_HID, _HID)), _full((1, _HID)),
        ],
        out_specs=pl.BlockSpec((_M, _HID), lambda i: (i, 0)),
        out_shape=jax.ShapeDtypeStruct((nwords, _HID), jnp.float32),
    )


def _conv_toeplitz(conv_w):
    kflat = jnp.transpose(conv_w, (2, 1, 0)).reshape(_W * _CDIM, _F)
    cols = [jnp.pad(kflat, ((_CDIM * t, _CDIM * (_NPOS - 1 - t)), (0, 0)))
            for t in range(_NPOS)]
    return jnp.concatenate(cols, axis=1)


def kernel(x, c, word_table, unk_emb, proj_w, char_table, conv_w, conv_b,
           tw0, tb0, tw1, tb1, gw0, gb0, gw1, gb1):
    bf16 = jnp.bfloat16
    xf = x.astype(jnp.int32).reshape(-1)
    cf = c.astype(jnp.int32).reshape(-1)
    wrows, crows = _get_sc_gather(_NWORDS)(word_table, xf, char_table, cf)
    out = _get_tc_dense(_NWORDS)(
        crows.reshape(_NWORDS, _WL * _CDIM), xf.reshape(-1, 1), wrows,
        unk_emb, proj_w.T.astype(bf16),
        _conv_toeplitz(conv_w).astype(bf16), conv_b.reshape(1, _F),
        gw0.T.astype(bf16), gb0.reshape(1, _HID),
        tw0.T.astype(bf16), tb0.reshape(1, _HID),
        gw1.T.astype(bf16), gb1.reshape(1, _HID),
        tw1.T.astype(bf16), tb1.reshape(1, _HID))
    return out.reshape(_B, _S, _HID)


# speedup vs baseline: 1.1001x; 1.0264x over previous
import functools

import jax
import jax.numpy as jnp
from jax import lax
from jax.experimental import pallas as pl
from jax.experimental.pallas import tpu as pltpu
from jax.experimental.pallas import tpu_sc as plsc

_B, _S, _WL = 32, 512, 16
_WDIM, _CDIM, _F, _W = 128, 64, 64, 5
_HID = _WDIM + _F
_NWORDS = _B * _S
_NPOS = _WL - _W + 1

_NC, _NS = 2, 16
_NWK = _NC * _NS
_CHUNK = 128


@functools.cache
def _get_sc_gather(nwords):
    nchars = nwords * _WL
    wch = nwords // (_NWK * _CHUNK)
    cch = nchars // (_NWK * _CHUNK)
    ngrp = cch // 4

    def body(wtbl, widx, ctbl, cidx, wout, cout,
             widx_v, cidx_v, wbufs, cbufs, wgs, wss, cgs, css):
        wid = lax.axis_index("s") * _NC + lax.axis_index("c")
        wbase = wid * (wch * _CHUNK)
        cbase = wid * (cch * _CHUNK)
        pltpu.sync_copy(widx.at[wid], widx_v)
        pltpu.sync_copy(cidx.at[wid], cidx_v)

        def wgather(j, b):
            return pltpu.make_async_copy(
                wtbl.at[widx_v.at[j]], wbufs.at[b], wgs.at[b])

        def wstore(j, b):
            return pltpu.make_async_copy(
                wbufs.at[b], wout.at[pl.ds(wbase + j * _CHUNK, _CHUNK)],
                wss.at[b])

        wgather(0, 0).start()
        wgather(1, 1).start()

        def cgather(j, b):
            return pltpu.make_async_copy(
                ctbl.at[cidx_v.at[j]], cbufs.at[b], cgs.at[b])

        def cstore(j, b):
            return pltpu.make_async_copy(
                cbufs.at[b], cout.at[pl.ds(cbase + j * _CHUNK, _CHUNK)],
                css.at[b])

        for b in range(4):
            cgather(b, b).start()
        for b in range(4):
            cgather(b, b).wait()
            cstore(b, b).start()
        for b in range(4):
            cgather(4 + b, 4 + b).start()

        def grp(g, carry):
            cur = 4 * (g % 2)
            oth = 4 * ((g + 1) % 2)
            for b in range(4):
                cstore(4 * (g - 1) + b, oth + b).wait()
                cgather(4 * (g + 1) + b, oth + b).start()
            for b in range(4):
                cgather(4 * g + b, cur + b).wait()
                cstore(4 * g + b, cur + b).start()
            return carry

        lax.fori_loop(1, ngrp - 1, grp, 0)

        glast = ngrp - 1
        gl = 4 * (glast % 2)
        for b in range(4):
            cgather(4 * glast + b, gl + b).wait()
            cstore(4 * glast + b, gl + b).start()
        for b in range(4):
            cstore(4 * (glast - 1) + b, (4 - gl) + b).wait()
            cstore(4 * glast + b, gl + b).wait()

        for j in range(wch):
            b = j % 2
            wgather(j, b).wait()
            wstore(j, b).start()
            if j + 2 < wch:
                wstore(j, b).wait()
                wgather(j + 2, b).start()
        for j in range(max(wch - 2, 0), wch):
            wstore(j, j % 2).wait()

    return pl.kernel(
        body,
        out_type=[
            jax.ShapeDtypeStruct((nwords, _WDIM), jnp.float32),
            jax.ShapeDtypeStruct((nchars, _CDIM), jnp.float32),
        ],
        mesh=plsc.VectorSubcoreMesh(core_axis_name="c", subcore_axis_name="s",
                                    num_cores=_NC, num_subcores=_NS),
        scratch_types=[
            pltpu.VMEM((wch, _CHUNK), jnp.int32),
            pltpu.VMEM((cch, _CHUNK), jnp.int32),
            pltpu.VMEM((2, _CHUNK, _WDIM), jnp.float32),
            pltpu.VMEM((8, _CHUNK, _CDIM), jnp.float32),
            pltpu.SemaphoreType.DMA((2,)),
            pltpu.SemaphoreType.DMA((2,)),
            pltpu.SemaphoreType.DMA((8,)),
            pltpu.SemaphoreType.DMA((8,)),
        ],
        compiler_params=pltpu.CompilerParams(use_tc_tiling_on_sc=False),
    )


_M = 512


def _tc_dense_body(ce_ref, x_ref, wr_ref, unk_ref, pwt_ref, wc_ref, cb_ref,
                   gwt0_ref, gb0_ref, twt0_ref, tb0_ref,
                   gwt1_ref, gb1_ref, twt1_ref, tb1_ref, o_ref):
    f32 = jnp.float32
    bf16 = jnp.bfloat16
    z = jnp.dot(ce_ref[...].astype(bf16), wc_ref[...],
                preferred_element_type=f32)
    cb = cb_ref[...]
    cm = jnp.maximum(z[:, 0:_F] + cb, 0.0)
    for t in range(1, _NPOS):
        cm = jnp.maximum(cm, jnp.maximum(z[:, t * _F:(t + 1) * _F] + cb, 0.0))
    mask = x_ref[...] == 1
    emb = jnp.where(mask, unk_ref[...], wr_ref[...])
    p = jnp.dot(emb.astype(bf16), pwt_ref[...], preferred_element_type=f32)
    h = jnp.concatenate([p, cm], axis=1)
    for gwt, gb, twt, tb in ((gwt0_ref, gb0_ref, twt0_ref, tb0_ref),
                             (gwt1_ref, gb1_ref, twt1_ref, tb1_ref)):
        hb = h.astype(bf16)
        g = jax.nn.sigmoid(jnp.dot(hb, gwt[...], preferred_element_type=f32)
                           + gb[...])
        t = jnp.maximum(jnp.dot(hb, twt[...], preferred_element_type=f32)
                        + tb[...], 0.0)
        h = g * t + (1.0 - g) * h
    o_ref[...] = h


def _full(shape):
    return pl.BlockSpec(shape, lambda i: (0, 0))


@functools.cache
def _get_tc_dense(nwords):
    return pl.pallas_call(
        _tc_dense_body,
        grid=(nwords // _M,),
        in_specs=[
            pl.BlockSpec((_M, _WL * _CDIM), lambda i: (i, 0)),
            pl.BlockSpec((_M, 1), lambda i: (i, 0)),
            pl.BlockSpec((_M, _WDIM), lambda i: (i, 0)),
            _full((1, _WDIM)),
            _full((_WDIM, _WDIM)),
            _full((_WL * _CDIM, _NPOS * _F)),
            _full((1, _F)),
            _full((_HID, _HID)), _full((1, _HID)),
            _full((_HID, _HID)), _full((1, _HID)),
            _full((_HID, _HID)), _full((1, _HID)),
            _full((_HID, _HID)), _full((1, _HID)),
        ],
        out_specs=pl.BlockSpec((_M, _HID), lambda i: (i, 0)),
        out_shape=jax.ShapeDtypeStruct((nwords, _HID), jnp.float32),
    )


def _conv_toeplitz(conv_w):
    kflat = jnp.transpose(conv_w, (2, 1, 0)).reshape(_W * _CDIM, _F)
    cols = [jnp.pad(kflat, ((_CDIM * t, _CDIM * (_NPOS - 1 - t)), (0, 0)))
            for t in range(_NPOS)]
    return jnp.concatenate(cols, axis=1)


def kernel(x, c, word_table, unk_emb, proj_w, char_table, conv_w, conv_b,
           tw0, tb0, tw1, tb1, gw0, gb0, gw1, gb1):
    bf16 = jnp.bfloat16
    xf = x.astype(jnp.int32).reshape(-1)
    cf = c.astype(jnp.int32).reshape(-1)
    wrows, crows = _get_sc_gather(_NWORDS)(
        word_table, xf.reshape(_NWK, -1, _CHUNK),
        char_table, cf.reshape(_NWK, -1, _CHUNK))
    out = _get_tc_dense(_NWORDS)(
        crows.reshape(_NWORDS, _WL * _CDIM), xf.reshape(-1, 1), wrows,
        unk_emb, proj_w.T.astype(bf16),
        _conv_toeplitz(conv_w).astype(bf16), conv_b.reshape(1, _F),
        gw0.T.astype(bf16), gb0.reshape(1, _HID),
        tw0.T.astype(bf16), tb0.reshape(1, _HID),
        gw1.T.astype(bf16), gb1.reshape(1, _HID),
        tw1.T.astype(bf16), tb1.reshape(1, _HID))
    return out.reshape(_B, _S, _HID)
